# baseline (speedup 1.0000x reference)
import functools

import jax
import jax.numpy as jnp
from jax import lax
from jax.experimental import pallas as pl
from jax.experimental.pallas import tpu as pltpu
from jax.experimental.pallas import tpu_sc as plsc

E = 8
K = 2
BLK = 128
NW = 32


def _router_body(x_ref, wg_ref, e1_ref, e2_ref, p1_ref, p2_ref):
    x = x_ref[...]
    wg = wg_ref[...]
    s = lax.dot_general(x, wg, (((1,), (1,)), ((), ())),
                        preferred_element_type=jnp.float32)
    t, e = s.shape
    col = lax.broadcasted_iota(jnp.int32, (t, e), 1)
    m1 = jnp.max(s, axis=1, keepdims=True)
    i1 = jnp.min(jnp.where(s == m1, col, e), axis=1, keepdims=True)
    s2 = jnp.where(col == i1, -jnp.inf, s)
    m2 = jnp.max(s2, axis=1, keepdims=True)
    i2 = jnp.min(jnp.where(s2 == m2, col, e), axis=1, keepdims=True)
    p1 = jax.nn.sigmoid(m1 - m2)
    e1_ref[...] = i1
    e2_ref[...] = i2
    p1_ref[...] = p1
    p2_ref[...] = 1.0 - p1


def _router(xf, wg):
    t = xf.shape[0]
    return pl.pallas_call(
        _router_body,
        out_shape=(
            jax.ShapeDtypeStruct((t, 1), jnp.int32),
            jax.ShapeDtypeStruct((t, 1), jnp.int32),
            jax.ShapeDtypeStruct((t, 1), jnp.float32),
            jax.ShapeDtypeStruct((t, 1), jnp.float32),
        ),
    )(xf, wg)


def _route_metadata(e1, e2, num_tiles):
    t = e1.shape[0]
    p = K * t
    pairs = jnp.stack([e1, e2], axis=1).reshape(p)
    onehot = (pairs[:, None] == jnp.arange(E, dtype=jnp.int32)[None, :])
    csum = jnp.cumsum(onehot.astype(jnp.int32), axis=0)
    counts = csum[-1]
    rank = jnp.take_along_axis(csum, pairs[:, None], axis=1)[:, 0] - 1
    aligned = ((counts + BLK - 1) // BLK) * BLK
    astart = jnp.cumsum(aligned) - aligned
    pos = jnp.take(astart, pairs) + rank
    pos1 = pos[0::2]
    pos2 = pos[1::2]
    bound = jnp.cumsum(aligned)
    tile_e = jnp.searchsorted(
        bound, jnp.arange(num_tiles, dtype=jnp.int32) * BLK, side='right')
    tile_e = jnp.minimum(tile_e, E - 1).astype(jnp.int32)
    return pos1.astype(jnp.int32), pos2.astype(jnp.int32), tile_e


def _make_dispatch(t, dwords, pad):
    per = t // NW
    mesh = plsc.VectorSubcoreMesh(core_axis_name="c", subcore_axis_name="s")

    @functools.partial(
        pl.kernel,
        out_type=(
            jax.ShapeDtypeStruct((pad, dwords), jnp.int32),
            jax.ShapeDtypeStruct((pad,), jnp.float32),
        ),
        mesh=mesh,
        scratch_types=[
            pltpu.VMEM((per,), jnp.int32),
            pltpu.VMEM((per,), jnp.int32),
            pltpu.VMEM((per, dwords), jnp.int32),
            pltpu.VMEM((per,), jnp.float32),
            pltpu.VMEM((per,), jnp.float32),
            pltpu.SemaphoreType.DMA,
        ],
    )
    def dispatch(x32_hbm, pos1_hbm, pos2_hbm, p1_hbm, p2_hbm,
                 xd_hbm, prob_hbm, idx1_v, idx2_v, rows_v, p1_v, p2_v, sem):
        wid = lax.axis_index("s") * 2 + lax.axis_index("c")
        base = wid * per
        pltpu.sync_copy(pos1_hbm.at[pl.ds(base, per)], idx1_v)
        pltpu.sync_copy(pos2_hbm.at[pl.ds(base, per)], idx2_v)
        pltpu.sync_copy(x32_hbm.at[pl.ds(base, per)], rows_v)
        pltpu.sync_copy(p1_hbm.at[pl.ds(base, per)], p1_v)
        pltpu.sync_copy(p2_hbm.at[pl.ds(base, per)], p2_v)
        c1 = pltpu.async_copy(rows_v, xd_hbm.at[idx1_v], sem)
        c2 = pltpu.async_copy(rows_v, xd_hbm.at[idx2_v], sem)
        c3 = pltpu.async_copy(p1_v, prob_hbm.at[idx1_v], sem)
        c4 = pltpu.async_copy(p2_v, prob_hbm.at[idx2_v], sem)
        c1.wait()
        c2.wait()
        c3.wait()
        c4.wait()

    return dispatch


def _make_combine(t, d, pad):
    per = t // NW
    ch = 32
    vregs = (ch * d) // 16
    mesh = plsc.VectorSubcoreMesh(core_axis_name="c", subcore_axis_name="s")

    @functools.partial(
        pl.kernel,
        out_type=jax.ShapeDtypeStruct((t, d), jnp.float32),
        mesh=mesh,
        scratch_types=[
            pltpu.VMEM((ch,), jnp.int32),
            pltpu.VMEM((ch,), jnp.int32),
            pltpu.VMEM((ch, d), jnp.float32),
            pltpu.VMEM((ch, d), jnp.float32),
            pltpu.SemaphoreType.DMA,
        ],
    )
    def combine(od_hbm, pos1_hbm, pos2_hbm, out_hbm,
                idx1_v, idx2_v, rows1_v, rows2_v, sem):
        wid = lax.axis_index("s") * 2 + lax.axis_index("c")
        base = wid * per
        slices_per_row = d // 16

        for c in range(per // ch):
            off = base + c * ch
            pltpu.sync_copy(pos1_hbm.at[pl.ds(off, ch)], idx1_v)
            pltpu.sync_copy(pos2_hbm.at[pl.ds(off, ch)], idx2_v)
            g1 = pltpu.async_copy(od_hbm.at[idx1_v], rows1_v, sem)
            g2 = pltpu.async_copy(od_hbm.at[idx2_v], rows2_v, sem)
            g1.wait()
            g2.wait()

            def body(i, _):
                r = i // slices_per_row
                v = (i % slices_per_row) * 16
                rows1_v[r, pl.ds(v, 16)] = (
                    rows1_v[r, pl.ds(v, 16)] + rows2_v[r, pl.ds(v, 16)])
                return 0

            lax.fori_loop(0, vregs, body, 0)
            pltpu.sync_copy(rows1_v, out_hbm.at[pl.ds(off, ch)])

    return combine


def _ffn_body(te_ref, xd_ref, w1_ref, w2_ref, w3_ref, pd_ref, od_ref):
    del te_ref
    xb = xd_ref[...]
    a = lax.dot_general(xb, w1_ref[0], (((1,), (1,)), ((), ())),
                        preferred_element_type=jnp.float32)
    b = lax.dot_general(xb, w2_ref[0], (((1,), (1,)), ((), ())),
                        preferred_element_type=jnp.float32)
    h = a * b
    h = h * jax.nn.sigmoid(h)
    hb = h.astype(jnp.bfloat16)
    o = lax.dot_general(hb, w3_ref[0], (((1,), (1,)), ((), ())),
                        preferred_element_type=jnp.float32)
    od_ref[...] = o * pd_ref[...]


def _ffn(tile_e, xd, w1, w2, w3, prob_d, num_tiles):
    pad, d = xd.shape
    h = w1.shape[1]
    grid_spec = pltpu.PrefetchScalarGridSpec(
        num_scalar_prefetch=1,
        grid=(num_tiles,),
        in_specs=[
            pl.BlockSpec((BLK, d), lambda i, te: (i, 0)),
            pl.BlockSpec((1, h, d), lambda i, te: (te[i], 0, 0)),
            pl.BlockSpec((1, h, d), lambda i, te: (te[i], 0, 0)),
            pl.BlockSpec((1, d, h), lambda i, te: (te[i], 0, 0)),
            pl.BlockSpec((BLK, 1), lambda i, te: (i, 0)),
        ],
        out_specs=pl.BlockSpec((BLK, d), lambda i, te: (i, 0)),
    )
    return pl.pallas_call(
        _ffn_body,
        grid_spec=grid_spec,
        out_shape=jax.ShapeDtypeStruct((pad, d), jnp.float32),
        compiler_params=pltpu.CompilerParams(
            dimension_semantics=("arbitrary",)),
    )(tile_e, xd, w1, w2, w3, prob_d)


def kernel(x, Wg, W1, W2, W3):
    b, s, d = x.shape
    t = b * s
    hid = W1.shape[1]
    pad = K * t + E * BLK
    num_tiles = pad // BLK

    xf = x.reshape(t, d)
    e1, e2, p1, p2 = _router(xf, Wg)
    pos1, pos2, tile_e = _route_metadata(e1[:, 0], e2[:, 0], num_tiles)

    xb16 = xf.astype(jnp.bfloat16)
    x32 = lax.bitcast_convert_type(xb16.reshape(t, d // 2, 2), jnp.int32)

    xd32, prob_d = _make_dispatch(t, d // 2, pad)(
        x32, pos1, pos2, p1.reshape(t), p2.reshape(t))
    xd = lax.bitcast_convert_type(xd32, jnp.bfloat16).reshape(pad, d)

    od = _ffn(tile_e, xd,
              W1.astype(jnp.bfloat16), W2.astype(jnp.bfloat16),
              W3.astype(jnp.bfloat16), prob_d.reshape(pad, 1), num_tiles)

    out = _make_combine(t, d, pad)(od, pos1, pos2)
    return out.reshape(b, s, d)


# speedup vs baseline: 1.6916x; 1.6916x over previous
import functools

import jax
import jax.numpy as jnp
from jax import lax
from jax.experimental import pallas as pl
from jax.experimental.pallas import tpu as pltpu
from jax.experimental.pallas import tpu_sc as plsc

E = 8
K = 2
BLK = 128
NW = 32


def _router_body(x_ref, wg_ref, e1_ref, e2_ref, p1_ref, p2_ref):
    x = x_ref[...]
    wg = wg_ref[...]
    s = lax.dot_general(x, wg, (((1,), (1,)), ((), ())),
                        preferred_element_type=jnp.float32)
    t, e = s.shape
    col = lax.broadcasted_iota(jnp.int32, (t, e), 1)
    m1 = jnp.max(s, axis=1, keepdims=True)
    i1 = jnp.min(jnp.where(s == m1, col, e), axis=1, keepdims=True)
    s2 = jnp.where(col == i1, -jnp.inf, s)
    m2 = jnp.max(s2, axis=1, keepdims=True)
    i2 = jnp.min(jnp.where(s2 == m2, col, e), axis=1, keepdims=True)
    p1 = jax.nn.sigmoid(m1 - m2)
    e1_ref[...] = i1
    e2_ref[...] = i2
    p1_ref[...] = p1
    p2_ref[...] = 1.0 - p1


def _router(xf, wg):
    t = xf.shape[0]
    return pl.pallas_call(
        _router_body,
        out_shape=(
            jax.ShapeDtypeStruct((t, 1), jnp.int32),
            jax.ShapeDtypeStruct((t, 1), jnp.int32),
            jax.ShapeDtypeStruct((t, 1), jnp.float32),
            jax.ShapeDtypeStruct((t, 1), jnp.float32),
        ),
    )(xf, wg)


def _route_metadata(e1, e2, num_tiles):
    t = e1.shape[0]
    p = K * t
    pairs = jnp.stack([e1, e2], axis=1).reshape(p)
    onehot = (pairs[:, None] == jnp.arange(E, dtype=jnp.int32)[None, :])
    csum = jnp.cumsum(onehot.astype(jnp.int32), axis=0)
    counts = csum[-1]
    rank = jnp.take_along_axis(csum, pairs[:, None], axis=1)[:, 0] - 1
    aligned = ((counts + BLK - 1) // BLK) * BLK
    astart = jnp.cumsum(aligned) - aligned
    pos = jnp.take(astart, pairs) + rank
    pos1 = pos[0::2]
    pos2 = pos[1::2]
    bound = jnp.cumsum(aligned)
    tile_e = jnp.searchsorted(
        bound, jnp.arange(num_tiles, dtype=jnp.int32) * BLK, side='right')
    tile_e = jnp.minimum(tile_e, E - 1).astype(jnp.int32)
    return pos1.astype(jnp.int32), pos2.astype(jnp.int32), tile_e


def _make_dispatch(t, d, pad):
    per = t // NW
    mesh = plsc.VectorSubcoreMesh(core_axis_name="c", subcore_axis_name="s")

    @functools.partial(
        pl.kernel,
        out_type=(
            jax.ShapeDtypeStruct((pad, d), jnp.float32),
            jax.ShapeDtypeStruct((pad,), jnp.float32),
        ),
        mesh=mesh,
        scratch_types=[
            pltpu.VMEM((per,), jnp.int32),
            pltpu.VMEM((per,), jnp.int32),
            pltpu.VMEM((per, d), jnp.float32),
            pltpu.VMEM((per,), jnp.float32),
            pltpu.VMEM((per,), jnp.float32),
            pltpu.SemaphoreType.DMA,
        ],
    )
    def dispatch(x_hbm, pos1_hbm, pos2_hbm, p1_hbm, p2_hbm,
                 xd_hbm, prob_hbm, idx1_v, idx2_v, rows_v, p1_v, p2_v, sem):
        wid = lax.axis_index("s") * 2 + lax.axis_index("c")
        base = wid * per
        pltpu.sync_copy(pos1_hbm.at[pl.ds(base, per)], idx1_v)
        pltpu.sync_copy(pos2_hbm.at[pl.ds(base, per)], idx2_v)
        pltpu.sync_copy(x_hbm.at[pl.ds(base, per)], rows_v)
        pltpu.sync_copy(p1_hbm.at[pl.ds(base, per)], p1_v)
        pltpu.sync_copy(p2_hbm.at[pl.ds(base, per)], p2_v)
        c1 = pltpu.async_copy(rows_v, xd_hbm.at[idx1_v], sem)
        c2 = pltpu.async_copy(rows_v, xd_hbm.at[idx2_v], sem)
        c3 = pltpu.async_copy(p1_v, prob_hbm.at[idx1_v], sem)
        c4 = pltpu.async_copy(p2_v, prob_hbm.at[idx2_v], sem)
        c1.wait()
        c2.wait()
        c3.wait()
        c4.wait()

    return dispatch


def _make_combine(t, d, pad):
    per = t // NW
    ch = 32
    vregs = (ch * d) // 16
    mesh = plsc.VectorSubcoreMesh(core_axis_name="c", subcore_axis_name="s")

    @functools.partial(
        pl.kernel,
        out_type=jax.ShapeDtypeStruct((t, d), jnp.float32),
        mesh=mesh,
        scratch_types=[
            pltpu.VMEM((ch,), jnp.int32),
            pltpu.VMEM((ch,), jnp.int32),
            pltpu.VMEM((ch, d), jnp.float32),
            pltpu.VMEM((ch, d), jnp.float32),
            pltpu.SemaphoreType.DMA,
        ],
    )
    def combine(od_hbm, pos1_hbm, pos2_hbm, out_hbm,
                idx1_v, idx2_v, rows1_v, rows2_v, sem):
        wid = lax.axis_index("s") * 2 + lax.axis_index("c")
        base = wid * per
        slices_per_row = d // 16

        for c in range(per // ch):
            off = base + c * ch
            pltpu.sync_copy(pos1_hbm.at[pl.ds(off, ch)], idx1_v)
            pltpu.sync_copy(pos2_hbm.at[pl.ds(off, ch)], idx2_v)
            g1 = pltpu.async_copy(od_hbm.at[idx1_v], rows1_v, sem)
            g2 = pltpu.async_copy(od_hbm.at[idx2_v], rows2_v, sem)
            g1.wait()
            g2.wait()

            def body(i, _):
                r = i // slices_per_row
                v = (i % slices_per_row) * 16
                rows1_v[r, pl.ds(v, 16)] = (
                    rows1_v[r, pl.ds(v, 16)] + rows2_v[r, pl.ds(v, 16)])
                return 0

            lax.fori_loop(0, vregs, body, 0)
            pltpu.sync_copy(rows1_v, out_hbm.at[pl.ds(off, ch)])

    return combine


def _ffn_body(te_ref, xd_ref, w1_ref, w2_ref, w3_ref, pd_ref, od_ref):
    del te_ref
    xb = xd_ref[...]
    a = lax.dot_general(xb, w1_ref[0], (((1,), (1,)), ((), ())),
                        preferred_element_type=jnp.float32)
    b = lax.dot_general(xb, w2_ref[0], (((1,), (1,)), ((), ())),
                        preferred_element_type=jnp.float32)
    h = a * b
    h = h * jax.nn.sigmoid(h)
    o = lax.dot_general(h, w3_ref[0], (((1,), (1,)), ((), ())),
                        preferred_element_type=jnp.float32)
    od_ref[...] = o * pd_ref[...]


def _ffn(tile_e, xd, w1, w2, w3, prob_d, num_tiles):
    pad, d = xd.shape
    h = w1.shape[1]
    grid_spec = pltpu.PrefetchScalarGridSpec(
        num_scalar_prefetch=1,
        grid=(num_tiles,),
        in_specs=[
            pl.BlockSpec((BLK, d), lambda i, te: (i, 0)),
            pl.BlockSpec((1, h, d), lambda i, te: (te[i], 0, 0)),
            pl.BlockSpec((1, h, d), lambda i, te: (te[i], 0, 0)),
            pl.BlockSpec((1, d, h), lambda i, te: (te[i], 0, 0)),
            pl.BlockSpec((BLK, 1), lambda i, te: (i, 0)),
        ],
        out_specs=pl.BlockSpec((BLK, d), lambda i, te: (i, 0)),
    )
    return pl.pallas_call(
        _ffn_body,
        grid_spec=grid_spec,
        out_shape=jax.ShapeDtypeStruct((pad, d), jnp.float32),
        compiler_params=pltpu.CompilerParams(
            dimension_semantics=("arbitrary",)),
    )(tile_e, xd, w1, w2, w3, prob_d)


def kernel(x, Wg, W1, W2, W3):
    b, s, d = x.shape
    t = b * s
    hid = W1.shape[1]
    pad = K * t + E * BLK
    num_tiles = pad // BLK

    xf = x.reshape(t, d)
    e1, e2, p1, p2 = _router(xf, Wg)
    pos1, pos2, tile_e = _route_metadata(e1[:, 0], e2[:, 0], num_tiles)

    xd, prob_d = _make_dispatch(t, d, pad)(
        xf, pos1, pos2, p1.reshape(t), p2.reshape(t))

    od = _ffn(tile_e, xd, W1, W2, W3, prob_d.reshape(pad, 1), num_tiles)

    out = _make_combine(t, d, pad)(od, pos1, pos2)
    return out.reshape(b, s, d)


# speedup vs baseline: 2.4331x; 1.4383x over previous
import functools

import jax
import jax.numpy as jnp
from jax import lax
from jax.experimental import pallas as pl
from jax.experimental.pallas import tpu as pltpu
from jax.experimental.pallas import tpu_sc as plsc

E = 8
K = 2
BLK = 256
NW = 32


def _router_body(x_ref, wg_ref, e1_ref, e2_ref, p1_ref, p2_ref):
    x = x_ref[...]
    wg = wg_ref[...]
    s = lax.dot_general(x, wg, (((1,), (1,)), ((), ())),
                        preferred_element_type=jnp.float32)
    t, e = s.shape
    col = lax.broadcasted_iota(jnp.int32, (t, e), 1)
    m1 = jnp.max(s, axis=1, keepdims=True)
    i1 = jnp.min(jnp.where(s == m1, col, e), axis=1, keepdims=True)
    s2 = jnp.where(col == i1, -jnp.inf, s)
    m2 = jnp.max(s2, axis=1, keepdims=True)
    i2 = jnp.min(jnp.where(s2 == m2, col, e), axis=1, keepdims=True)
    p1 = jax.nn.sigmoid(m1 - m2)
    e1_ref[...] = i1
    e2_ref[...] = i2
    p1_ref[...] = p1
    p2_ref[...] = 1.0 - p1


def _router(xf, wg):
    t = xf.shape[0]
    return pl.pallas_call(
        _router_body,
        out_shape=(
            jax.ShapeDtypeStruct((t, 1), jnp.int32),
            jax.ShapeDtypeStruct((t, 1), jnp.int32),
            jax.ShapeDtypeStruct((t, 1), jnp.float32),
            jax.ShapeDtypeStruct((t, 1), jnp.float32),
        ),
    )(xf, wg)


def _route_metadata(e1, e2, num_tiles):
    t = e1.shape[0]
    p = K * t
    pairs = jnp.stack([e1, e2], axis=1).reshape(p)
    onehot = (pairs[:, None] == jnp.arange(E, dtype=jnp.int32)[None, :])
    csum = jnp.cumsum(onehot.astype(jnp.int32), axis=0)
    counts = csum[-1]
    rank = jnp.take_along_axis(csum, pairs[:, None], axis=1)[:, 0] - 1
    aligned = ((counts + BLK - 1) // BLK) * BLK
    astart = jnp.cumsum(aligned) - aligned
    pos = jnp.take(astart, pairs) + rank
    pos1 = pos[0::2]
    pos2 = pos[1::2]
    bound = jnp.cumsum(aligned)
    tile_e = jnp.searchsorted(
        bound, jnp.arange(num_tiles, dtype=jnp.int32) * BLK, side='right')
    return pos1.astype(jnp.int32), pos2.astype(jnp.int32), tile_e.astype(jnp.int32)


def _make_dispatch(t, d, pad):
    per = t // NW
    mesh = plsc.VectorSubcoreMesh(core_axis_name="c", subcore_axis_name="s")

    @functools.partial(
        pl.kernel,
        out_type=(
            jax.ShapeDtypeStruct((pad, d), jnp.float32),
            jax.ShapeDtypeStruct((pad,), jnp.float32),
        ),
        mesh=mesh,
        scratch_types=[
            pltpu.VMEM((per,), jnp.int32),
            pltpu.VMEM((per,), jnp.int32),
            pltpu.VMEM((per, d), jnp.float32),
            pltpu.VMEM((per,), jnp.float32),
            pltpu.VMEM((per,), jnp.float32),
            pltpu.SemaphoreType.DMA,
        ],
    )
    def dispatch(x_hbm, pos1_hbm, pos2_hbm, p1_hbm, p2_hbm,
                 xd_hbm, prob_hbm, idx1_v, idx2_v, rows_v, p1_v, p2_v, sem):
        wid = lax.axis_index("s") * 2 + lax.axis_index("c")
        base = wid * per
        pltpu.sync_copy(pos1_hbm.at[pl.ds(base, per)], idx1_v)
        pltpu.sync_copy(pos2_hbm.at[pl.ds(base, per)], idx2_v)
        pltpu.sync_copy(x_hbm.at[pl.ds(base, per)], rows_v)
        pltpu.sync_copy(p1_hbm.at[pl.ds(base, per)], p1_v)
        pltpu.sync_copy(p2_hbm.at[pl.ds(base, per)], p2_v)
        c1 = pltpu.async_copy(rows_v, xd_hbm.at[idx1_v], sem)
        c2 = pltpu.async_copy(rows_v, xd_hbm.at[idx2_v], sem)
        c3 = pltpu.async_copy(p1_v, prob_hbm.at[idx1_v], sem)
        c4 = pltpu.async_copy(p2_v, prob_hbm.at[idx2_v], sem)
        c1.wait()
        c2.wait()
        c3.wait()
        c4.wait()

    return dispatch


def _make_combine(t, d, pad):
    per = t // NW
    ch = 32
    vregs = (ch * d) // 16
    mesh = plsc.VectorSubcoreMesh(core_axis_name="c", subcore_axis_name="s")

    @functools.partial(
        pl.kernel,
        out_type=jax.ShapeDtypeStruct((t, d), jnp.float32),
        mesh=mesh,
        scratch_types=[
            pltpu.VMEM((ch,), jnp.int32),
            pltpu.VMEM((ch,), jnp.int32),
            pltpu.VMEM((ch, d), jnp.float32),
            pltpu.VMEM((ch, d), jnp.float32),
            pltpu.SemaphoreType.DMA,
        ],
    )
    def combine(od_hbm, pos1_hbm, pos2_hbm, out_hbm,
                idx1_v, idx2_v, rows1_v, rows2_v, sem):
        wid = lax.axis_index("s") * 2 + lax.axis_index("c")
        base = wid * per
        slices_per_row = d // 16

        for c in range(per // ch):
            off = base + c * ch
            pltpu.sync_copy(pos1_hbm.at[pl.ds(off, ch)], idx1_v)
            pltpu.sync_copy(pos2_hbm.at[pl.ds(off, ch)], idx2_v)
            g1 = pltpu.async_copy(od_hbm.at[idx1_v], rows1_v, sem)
            g2 = pltpu.async_copy(od_hbm.at[idx2_v], rows2_v, sem)
            g1.wait()
            g2.wait()

            def body(r, _):
                for v in range(slices_per_row):
                    sl = pl.ds(v * 16, 16)
                    rows1_v[r, sl] = rows1_v[r, sl] + rows2_v[r, sl]
                return 0

            lax.fori_loop(0, ch, body, 0)
            pltpu.sync_copy(rows1_v, out_hbm.at[pl.ds(off, ch)])

    return combine


def _ffn_body(te_ref, xd_ref, w1_ref, w2_ref, w3_ref, pd_ref, od_ref):
    @pl.when(te_ref[pl.program_id(0)] < E)
    def _():
        xb = xd_ref[...]
        a = lax.dot_general(xb, w1_ref[0], (((1,), (1,)), ((), ())),
                            preferred_element_type=jnp.float32)
        b = lax.dot_general(xb, w2_ref[0], (((1,), (1,)), ((), ())),
                            preferred_element_type=jnp.float32)
        h = a * b
        h = h * jax.nn.sigmoid(h)
        o = lax.dot_general(h, w3_ref[0], (((1,), (1,)), ((), ())),
                            preferred_element_type=jnp.float32)
        od_ref[...] = o * pd_ref[...]


def _ffn(tile_e, xd, w1, w2, w3, prob_d, num_tiles):
    pad, d = xd.shape
    h = w1.shape[1]
    grid_spec = pltpu.PrefetchScalarGridSpec(
        num_scalar_prefetch=1,
        grid=(num_tiles,),
        in_specs=[
            pl.BlockSpec((BLK, d), lambda i, te: (i, 0)),
            pl.BlockSpec((1, h, d), lambda i, te: (jnp.minimum(te[i], E - 1), 0, 0)),
            pl.BlockSpec((1, h, d), lambda i, te: (jnp.minimum(te[i], E - 1), 0, 0)),
            pl.BlockSpec((1, d, h), lambda i, te: (jnp.minimum(te[i], E - 1), 0, 0)),
            pl.BlockSpec((BLK, 1), lambda i, te: (i, 0)),
        ],
        out_specs=pl.BlockSpec((BLK, d), lambda i, te: (i, 0)),
    )
    return pl.pallas_call(
        _ffn_body,
        grid_spec=grid_spec,
        out_shape=jax.ShapeDtypeStruct((pad, d), jnp.float32),
        compiler_params=pltpu.CompilerParams(
            dimension_semantics=("arbitrary",)),
    )(tile_e, xd, w1, w2, w3, prob_d)


def kernel(x, Wg, W1, W2, W3):
    b, s, d = x.shape
    t = b * s
    hid = W1.shape[1]
    pad = K * t + E * BLK
    num_tiles = pad // BLK

    xf = x.reshape(t, d)
    e1, e2, p1, p2 = _router(xf, Wg)
    pos1, pos2, tile_e = _route_metadata(e1[:, 0], e2[:, 0], num_tiles)

    xd, prob_d = _make_dispatch(t, d, pad)(
        xf, pos1, pos2, p1.reshape(t), p2.reshape(t))

    od = _ffn(tile_e, xd, W1, W2, W3, prob_d.reshape(pad, 1), num_tiles)

    out = _make_combine(t, d, pad)(od, pos1, pos2)
    return out.reshape(b, s, d)


# speedup vs baseline: 2.4751x; 1.0173x over previous
import functools

import jax
import jax.numpy as jnp
from jax import lax
from jax.experimental import pallas as pl
from jax.experimental.pallas import tpu as pltpu
from jax.experimental.pallas import tpu_sc as plsc

E = 8
K = 2
BLK = 256
NW = 32


def _cumsum_sublanes(v, n):
    k = 1
    while k < n:
        z = jnp.zeros((k,) + v.shape[1:], v.dtype)
        v = v + jnp.concatenate([z, v[:-k]], axis=0)
        k *= 2
    return v


def _router_body(x_ref, wg_ref, pos1_ref, pos2_ref, p1_ref, p2_ref, te_ref):
    x = x_ref[...]
    wg = wg_ref[...]
    s = lax.dot_general(x, wg, (((1,), (1,)), ((), ())),
                        preferred_element_type=jnp.float32)
    t, e = s.shape
    col = lax.broadcasted_iota(jnp.int32, (t, e), 1)
    m1 = jnp.max(s, axis=1, keepdims=True)
    i1 = jnp.min(jnp.where(s == m1, col, e), axis=1, keepdims=True)
    s2 = jnp.where(col == i1, -jnp.inf, s)
    m2 = jnp.max(s2, axis=1, keepdims=True)
    i2 = jnp.min(jnp.where(s2 == m2, col, e), axis=1, keepdims=True)
    p1 = jax.nn.sigmoid(m1 - m2)
    p1_ref[...] = p1
    p2_ref[...] = 1.0 - p1

    oh1 = (col == i1).astype(jnp.int32)
    oh2 = (col == i2).astype(jnp.int32)
    oh = oh1 + oh2
    csum = _cumsum_sublanes(oh, t)
    counts = lax.slice(csum, (t - 1, 0), (t, e))
    csum_ex = csum - oh
    aligned = ((counts + BLK - 1) // BLK) * BLK
    bound = _cumsum_sublanes(aligned.reshape(e, 1), e).reshape(1, e)
    astart = bound - aligned
    rank1 = jnp.sum(csum_ex * oh1, axis=1, keepdims=True)
    rank2 = jnp.sum(csum_ex * oh2, axis=1, keepdims=True)
    a1 = jnp.sum(astart * oh1, axis=1, keepdims=True)
    a2 = jnp.sum(astart * oh2, axis=1, keepdims=True)
    pos1_ref[...] = a1 + rank1
    pos2_ref[...] = a2 + rank2

    nt = te_ref.shape[0]
    starts = lax.broadcasted_iota(jnp.int32, (nt, 1), 0) * BLK
    te_ref[...] = jnp.sum((bound <= starts).astype(jnp.int32),
                          axis=1, keepdims=True)


def _router(xf, wg, num_tiles):
    t = xf.shape[0]
    return pl.pallas_call(
        _router_body,
        out_shape=(
            jax.ShapeDtypeStruct((t, 1), jnp.int32),
            jax.ShapeDtypeStruct((t, 1), jnp.int32),
            jax.ShapeDtypeStruct((t, 1), jnp.float32),
            jax.ShapeDtypeStruct((t, 1), jnp.float32),
            jax.ShapeDtypeStruct((num_tiles, 1), jnp.int32),
        ),
    )(xf, wg)


def _make_dispatch(t, d, pad):
    per = t // NW
    mesh = plsc.VectorSubcoreMesh(core_axis_name="c", subcore_axis_name="s")

    @functools.partial(
        pl.kernel,
        out_type=(
            jax.ShapeDtypeStruct((pad, d), jnp.float32),
            jax.ShapeDtypeStruct((pad,), jnp.float32),
        ),
        mesh=mesh,
        scratch_types=[
            pltpu.VMEM((per,), jnp.int32),
            pltpu.VMEM((per,), jnp.int32),
            pltpu.VMEM((per, d), jnp.float32),
            pltpu.VMEM((per,), jnp.float32),
            pltpu.VMEM((per,), jnp.float32),
            pltpu.SemaphoreType.DMA,
        ],
    )
    def dispatch(x_hbm, pos1_hbm, pos2_hbm, p1_hbm, p2_hbm,
                 xd_hbm, prob_hbm, idx1_v, idx2_v, rows_v, p1_v, p2_v, sem):
        wid = lax.axis_index("s") * 2 + lax.axis_index("c")
        base = wid * per
        sl = pl.ds(base, per)
        reads = [pltpu.async_copy(pos1_hbm.at[sl], idx1_v, sem),
                 pltpu.async_copy(pos2_hbm.at[sl], idx2_v, sem),
                 pltpu.async_copy(x_hbm.at[sl], rows_v, sem),
                 pltpu.async_copy(p1_hbm.at[sl], p1_v, sem),
                 pltpu.async_copy(p2_hbm.at[sl], p2_v, sem)]
        for r in reads:
            r.wait()
        writes = [pltpu.async_copy(rows_v, xd_hbm.at[idx1_v], sem),
                  pltpu.async_copy(rows_v, xd_hbm.at[idx2_v], sem),
                  pltpu.async_copy(p1_v, prob_hbm.at[idx1_v], sem),
                  pltpu.async_copy(p2_v, prob_hbm.at[idx2_v], sem)]
        for w in writes:
            w.wait()

    return dispatch


def _make_combine(t, d, pad):
    per = t // NW
    ch = 32
    vregs = (ch * d) // 16
    mesh = plsc.VectorSubcoreMesh(core_axis_name="c", subcore_axis_name="s")

    @functools.partial(
        pl.kernel,
        out_type=jax.ShapeDtypeStruct((t, d), jnp.float32),
        mesh=mesh,
        scratch_types=[
            pltpu.VMEM((ch,), jnp.int32),
            pltpu.VMEM((ch,), jnp.int32),
            pltpu.VMEM((ch, d), jnp.float32),
            pltpu.VMEM((ch, d), jnp.float32),
            pltpu.SemaphoreType.DMA,
        ],
    )
    def combine(od_hbm, pos1_hbm, pos2_hbm, out_hbm,
                idx1_v, idx2_v, rows1_v, rows2_v, sem):
        wid = lax.axis_index("s") * 2 + lax.axis_index("c")
        base = wid * per
        slices_per_row = d // 16

        for c in range(per // ch):
            off = base + c * ch
            pltpu.sync_copy(pos1_hbm.at[pl.ds(off, ch)], idx1_v)
            pltpu.sync_copy(pos2_hbm.at[pl.ds(off, ch)], idx2_v)
            g1 = pltpu.async_copy(od_hbm.at[idx1_v], rows1_v, sem)
            g2 = pltpu.async_copy(od_hbm.at[idx2_v], rows2_v, sem)
            g1.wait()
            g2.wait()

            def body(r, _):
                for v in range(slices_per_row):
                    sl = pl.ds(v * 16, 16)
                    rows1_v[r, sl] = rows1_v[r, sl] + rows2_v[r, sl]
                return 0

            lax.fori_loop(0, ch, body, 0)
            pltpu.sync_copy(rows1_v, out_hbm.at[pl.ds(off, ch)])

    return combine


def _ffn_body(te_ref, xd_ref, w1_ref, w2_ref, w3_ref, pd_ref, od_ref):
    @pl.when(te_ref[pl.program_id(0)] < E)
    def _():
        xb = xd_ref[...]
        a = lax.dot_general(xb, w1_ref[0], (((1,), (1,)), ((), ())),
                            preferred_element_type=jnp.float32)
        b = lax.dot_general(xb, w2_ref[0], (((1,), (1,)), ((), ())),
                            preferred_element_type=jnp.float32)
        h = a * b
        h = h * jax.nn.sigmoid(h)
        o = lax.dot_general(h, w3_ref[0], (((1,), (1,)), ((), ())),
                            preferred_element_type=jnp.float32)
        od_ref[...] = o * pd_ref[...]


def _ffn(tile_e, xd, w1, w2, w3, prob_d, num_tiles):
    pad, d = xd.shape
    h = w1.shape[1]
    grid_spec = pltpu.PrefetchScalarGridSpec(
        num_scalar_prefetch=1,
        grid=(num_tiles,),
        in_specs=[
            pl.BlockSpec((BLK, d), lambda i, te: (i, 0)),
            pl.BlockSpec((1, h, d), lambda i, te: (jnp.minimum(te[i], E - 1), 0, 0)),
            pl.BlockSpec((1, h, d), lambda i, te: (jnp.minimum(te[i], E - 1), 0, 0)),
            pl.BlockSpec((1, d, h), lambda i, te: (jnp.minimum(te[i], E - 1), 0, 0)),
            pl.BlockSpec((BLK, 1), lambda i, te: (i, 0)),
        ],
        out_specs=pl.BlockSpec((BLK, d), lambda i, te: (i, 0)),
    )
    return pl.pallas_call(
        _ffn_body,
        grid_spec=grid_spec,
        out_shape=jax.ShapeDtypeStruct((pad, d), jnp.float32),
        compiler_params=pltpu.CompilerParams(
            dimension_semantics=("arbitrary",)),
    )(tile_e, xd, w1, w2, w3, prob_d)


def kernel(x, Wg, W1, W2, W3):
    b, s, d = x.shape
    t = b * s
    hid = W1.shape[1]
    pad = K * t + E * BLK
    num_tiles = pad // BLK

    xf = x.reshape(t, d)
    pos1, pos2, p1, p2, tile_e = _router(xf, Wg, num_tiles)
    pos1 = pos1.reshape(t)
    pos2 = pos2.reshape(t)

    xd, prob_d = _make_dispatch(t, d, pad)(
        xf, pos1, pos2, p1.reshape(t), p2.reshape(t))

    od = _ffn(tile_e.reshape(num_tiles), xd, W1, W2, W3,
              prob_d.reshape(pad, 1), num_tiles)

    out = _make_combine(t, d, pad)(od, pos1, pos2)
    return out.reshape(b, s, d)


# speedup vs baseline: 2.7770x; 1.1220x over previous
import functools

import jax
import jax.numpy as jnp
from jax import lax
from jax.experimental import pallas as pl
from jax.experimental.pallas import tpu as pltpu
from jax.experimental.pallas import tpu_sc as plsc

E = 8
K = 2
BLK = 256
NW = 32


def _cumsum_sublanes(v, n):
    k = 1
    while k < n:
        z = jnp.zeros((k,) + v.shape[1:], v.dtype)
        v = v + jnp.concatenate([z, v[:-k]], axis=0)
        k *= 2
    return v


def _router_body(x_ref, wg_ref, pos1_ref, pos2_ref, p1_ref, p2_ref, te_ref,
                 run_ref, nxt_ref):
    x = x_ref[...]
    wg = wg_ref[...]
    s = lax.dot_general(x, wg, (((1,), (1,)), ((), ())),
                        preferred_element_type=jnp.float32)
    t, e = s.shape
    col = lax.broadcasted_iota(jnp.int32, (t, e), 1)
    m1 = jnp.max(s, axis=1, keepdims=True)
    i1 = jnp.min(jnp.where(s == m1, col, e), axis=1, keepdims=True)
    s2 = jnp.where(col == i1, -jnp.inf, s)
    m2 = jnp.max(s2, axis=1, keepdims=True)
    i2 = jnp.min(jnp.where(s2 == m2, col, e), axis=1, keepdims=True)
    p1 = jax.nn.sigmoid(m1 - m2)
    p1_ref[...] = p1
    p2_ref[...] = 1.0 - p1

    oh1 = (col == i1).astype(jnp.int32)
    oh2 = (col == i2).astype(jnp.int32)
    oh = oh1 + oh2
    csum = _cumsum_sublanes(oh, t)
    counts = lax.slice(csum, (t - 1, 0), (t, e))
    csum_ex = csum - oh
    aligned = ((counts + BLK - 1) // BLK) * BLK
    bound = _cumsum_sublanes(aligned.reshape(e, 1), e).reshape(1, e)
    astart = bound - aligned
    rank1 = jnp.sum(csum_ex * oh1, axis=1, keepdims=True)
    rank2 = jnp.sum(csum_ex * oh2, axis=1, keepdims=True)
    a1 = jnp.sum(astart * oh1, axis=1, keepdims=True)
    a2 = jnp.sum(astart * oh2, axis=1, keepdims=True)
    pos1_ref[...] = a1 + rank1
    pos2_ref[...] = a2 + rank2

    nt = te_ref.shape[0]
    starts = lax.broadcasted_iota(jnp.int32, (nt, 1), 0) * BLK
    te = jnp.sum((bound <= starts).astype(jnp.int32), axis=1, keepdims=True)
    te_ref[...] = te
    erow = lax.broadcasted_iota(jnp.int32, (1, e), 1)
    nonempty = (counts > 0).astype(jnp.int32)
    run_ref[...] = jnp.sum(jnp.where(erow < te, nonempty, 0),
                           axis=1, keepdims=True)
    nxt_ref[...] = jnp.min(jnp.where((erow > te) & (nonempty > 0), erow, e),
                           axis=1, keepdims=True)


def _router(xf, wg, num_tiles):
    t = xf.shape[0]
    return pl.pallas_call(
        _router_body,
        out_shape=(
            jax.ShapeDtypeStruct((t, 1), jnp.int32),
            jax.ShapeDtypeStruct((t, 1), jnp.int32),
            jax.ShapeDtypeStruct((t, 1), jnp.float32),
            jax.ShapeDtypeStruct((t, 1), jnp.float32),
            jax.ShapeDtypeStruct((num_tiles, 1), jnp.int32),
            jax.ShapeDtypeStruct((num_tiles, 1), jnp.int32),
            jax.ShapeDtypeStruct((num_tiles, 1), jnp.int32),
        ),
    )(xf, wg)


def _make_dispatch(t, d, pad):
    per = t // NW
    mesh = plsc.VectorSubcoreMesh(core_axis_name="c", subcore_axis_name="s")

    @functools.partial(
        pl.kernel,
        out_type=(
            jax.ShapeDtypeStruct((pad, d), jnp.float32),
            jax.ShapeDtypeStruct((pad,), jnp.float32),
        ),
        mesh=mesh,
        scratch_types=[
            pltpu.VMEM((per,), jnp.int32),
            pltpu.VMEM((per,), jnp.int32),
            pltpu.VMEM((per, d), jnp.float32),
            pltpu.VMEM((per,), jnp.float32),
            pltpu.VMEM((per,), jnp.float32),
            pltpu.SemaphoreType.DMA,
        ],
    )
    def dispatch(x_hbm, pos1_hbm, pos2_hbm, p1_hbm, p2_hbm,
                 xd_hbm, prob_hbm, idx1_v, idx2_v, rows_v, p1_v, p2_v, sem):
        wid = lax.axis_index("s") * 2 + lax.axis_index("c")
        base = wid * per
        sl = pl.ds(base, per)
        reads = [pltpu.async_copy(pos1_hbm.at[sl], idx1_v, sem),
                 pltpu.async_copy(pos2_hbm.at[sl], idx2_v, sem),
                 pltpu.async_copy(x_hbm.at[sl], rows_v, sem),
                 pltpu.async_copy(p1_hbm.at[sl], p1_v, sem),
                 pltpu.async_copy(p2_hbm.at[sl], p2_v, sem)]
        for r in reads:
            r.wait()
        writes = [pltpu.async_copy(rows_v, xd_hbm.at[idx1_v], sem),
                  pltpu.async_copy(rows_v, xd_hbm.at[idx2_v], sem),
                  pltpu.async_copy(p1_v, prob_hbm.at[idx1_v], sem),
                  pltpu.async_copy(p2_v, prob_hbm.at[idx2_v], sem)]
        for w in writes:
            w.wait()

    return dispatch


def _make_combine(t, d, pad):
    per = t // NW
    ch = 32
    vregs = (ch * d) // 16
    mesh = plsc.VectorSubcoreMesh(core_axis_name="c", subcore_axis_name="s")

    @functools.partial(
        pl.kernel,
        out_type=jax.ShapeDtypeStruct((t, d), jnp.float32),
        mesh=mesh,
        scratch_types=[
            pltpu.VMEM((ch,), jnp.int32),
            pltpu.VMEM((ch,), jnp.int32),
            pltpu.VMEM((ch, d), jnp.float32),
            pltpu.VMEM((ch, d), jnp.float32),
            pltpu.SemaphoreType.DMA,
        ],
    )
    def combine(od_hbm, pos1_hbm, pos2_hbm, out_hbm,
                idx1_v, idx2_v, rows1_v, rows2_v, sem):
        wid = lax.axis_index("s") * 2 + lax.axis_index("c")
        base = wid * per
        slices_per_row = d // 16

        for c in range(per // ch):
            off = base + c * ch
            pltpu.sync_copy(pos1_hbm.at[pl.ds(off, ch)], idx1_v)
            pltpu.sync_copy(pos2_hbm.at[pl.ds(off, ch)], idx2_v)
            g1 = pltpu.async_copy(od_hbm.at[idx1_v], rows1_v, sem)
            g2 = pltpu.async_copy(od_hbm.at[idx2_v], rows2_v, sem)
            g1.wait()
            g2.wait()

            def body(r, _):
                for v in range(slices_per_row):
                    sl = pl.ds(v * 16, 16)
                    rows1_v[r, sl] = rows1_v[r, sl] + rows2_v[r, sl]
                return 0

            lax.fori_loop(0, ch, body, 0)
            pltpu.sync_copy(rows1_v, out_hbm.at[pl.ds(off, ch)])

    return combine


def _ffn_body(te_ref, run_ref, nxt_ref, xd_ref, w1_hbm, w2_hbm, w3_hbm,
              pd_ref, od_ref, w1b, w2b, w3b, sems):
    i = pl.program_id(0)
    te = te_ref[i]
    valid = te < E
    run = run_ref[i]
    slot = jnp.bitwise_and(run, 1)
    nxt = nxt_ref[i]
    prev = te_ref[jnp.maximum(i - 1, 0)]
    run_first = jnp.logical_and(valid, jnp.logical_or(i == 0, te != prev))

    def fire3(e, k):
        pltpu.make_async_copy(w1_hbm.at[e], w1b.at[k], sems.at[k, 0]).start()
        pltpu.make_async_copy(w2_hbm.at[e], w2b.at[k], sems.at[k, 1]).start()
        pltpu.make_async_copy(w3_hbm.at[e], w3b.at[k], sems.at[k, 2]).start()

    def wait3(k):
        pltpu.make_async_copy(w1_hbm.at[0], w1b.at[k], sems.at[k, 0]).wait()
        pltpu.make_async_copy(w2_hbm.at[0], w2b.at[k], sems.at[k, 1]).wait()
        pltpu.make_async_copy(w3_hbm.at[0], w3b.at[k], sems.at[k, 2]).wait()

    @pl.when(i == 0)
    def _():
        fire3(te, 0)
    @pl.when(jnp.logical_and(i == 0, nxt < E))
    def _():
        fire3(nxt, 1)
    later_first = jnp.logical_and(run_first, i > 0)
    fire_next = jnp.logical_and(later_first, nxt < E)
    @pl.when(jnp.logical_and(fire_next, slot == 0))
    def _():
        fire3(nxt, 1)
    @pl.when(jnp.logical_and(fire_next, slot == 1))
    def _():
        fire3(nxt, 0)
    @pl.when(jnp.logical_and(run_first, slot == 0))
    def _():
        wait3(0)
    @pl.when(jnp.logical_and(run_first, slot == 1))
    def _():
        wait3(1)

    @pl.when(valid)
    def _():
        xb = xd_ref[...]
        w1 = w1b[slot]
        w2 = w2b[slot]
        w3 = w3b[slot]
        a = lax.dot_general(xb, w1, (((1,), (1,)), ((), ())),
                            preferred_element_type=jnp.float32)
        b = lax.dot_general(xb, w2, (((1,), (1,)), ((), ())),
                            preferred_element_type=jnp.float32)
        h = a * b
        h = h * jax.nn.sigmoid(h)
        o = lax.dot_general(h, w3, (((1,), (1,)), ((), ())),
                            preferred_element_type=jnp.float32)
        od_ref[...] = o * pd_ref[...]


def _ffn(tile_e, tile_run, tile_nxt, xd, w1, w2, w3, prob_d, num_tiles):
    pad, d = xd.shape
    h = w1.shape[1]
    grid_spec = pltpu.PrefetchScalarGridSpec(
        num_scalar_prefetch=3,
        grid=(num_tiles,),
        in_specs=[
            pl.BlockSpec((BLK, d), lambda i, te, rn, nx: (i, 0)),
            pl.BlockSpec(memory_space=pl.ANY),
            pl.BlockSpec(memory_space=pl.ANY),
            pl.BlockSpec(memory_space=pl.ANY),
            pl.BlockSpec((BLK, 1), lambda i, te, rn, nx: (i, 0)),
        ],
        out_specs=pl.BlockSpec((BLK, d), lambda i, te, rn, nx: (i, 0)),
        scratch_shapes=[
            pltpu.VMEM((2, h, d), jnp.float32),
            pltpu.VMEM((2, h, d), jnp.float32),
            pltpu.VMEM((2, d, h), jnp.float32),
            pltpu.SemaphoreType.DMA((2, 3)),
        ],
    )
    return pl.pallas_call(
        _ffn_body,
        grid_spec=grid_spec,
        out_shape=jax.ShapeDtypeStruct((pad, d), jnp.float32),
        compiler_params=pltpu.CompilerParams(
            dimension_semantics=("arbitrary",)),
    )(tile_e, tile_run, tile_nxt, xd, w1, w2, w3, prob_d)


def kernel(x, Wg, W1, W2, W3):
    b, s, d = x.shape
    t = b * s
    hid = W1.shape[1]
    pad = K * t + E * BLK
    num_tiles = pad // BLK

    xf = x.reshape(t, d)
    pos1, pos2, p1, p2, tile_e, tile_run, tile_nxt = _router(xf, Wg, num_tiles)
    pos1 = pos1.reshape(t)
    pos2 = pos2.reshape(t)

    xd, prob_d = _make_dispatch(t, d, pad)(
        xf, pos1, pos2, p1.reshape(t), p2.reshape(t))

    od = _ffn(tile_e.reshape(num_tiles), tile_run.reshape(num_tiles),
              tile_nxt.reshape(num_tiles), xd, W1, W2, W3,
              prob_d.reshape(pad, 1), num_tiles)

    out = _make_combine(t, d, pad)(od, pos1, pos2)
    return out.reshape(b, s, d)
